# baseline (speedup 1.0000x reference)
import functools
import math

import jax
import jax.numpy as jnp
from jax import lax
from jax.experimental import pallas as pl
from jax.experimental.pallas import tpu as pltpu
from jax.experimental.pallas import tpu_sc as plsc

D_MODEL = 64
SCALE = math.sqrt(D_MODEL)

_info = plsc.get_sparse_core_info()
_NC = _info.num_cores
_NS = _info.num_subcores
_L = _info.num_lanes
_NW = _NC * _NS

CHUNK = 512


@functools.partial(jax.jit, static_argnums=())
def kernel(x, lut):
    B = x.shape[0] * x.shape[1]
    xf = x.reshape(B).astype(jnp.int32)
    b_per_w = B // _NW
    n_chunks = b_per_w // CHUNK

    mesh = plsc.VectorSubcoreMesh(core_axis_name="c", subcore_axis_name="s")

    @functools.partial(
        pl.kernel,
        mesh=mesh,
        out_type=jax.ShapeDtypeStruct((B, D_MODEL), jnp.float32),
        scratch_types=[
            pltpu.VMEM((CHUNK,), jnp.int32),
            pltpu.VMEM((CHUNK, D_MODEL), jnp.float32),
            pltpu.SemaphoreType.DMA,
        ],
        compiler_params=pltpu.CompilerParams(use_tc_tiling_on_sc=False),
    )
    def gather_scale(idx_hbm, table_hbm, out_hbm, idx_v, rows_v, sem):
        wid = lax.axis_index("s") * _NC + lax.axis_index("c")
        base = wid * b_per_w

        def chunk_body(g, carry):
            start = base + g * CHUNK
            pltpu.sync_copy(idx_hbm.at[pl.ds(start, CHUNK)], idx_v)
            pltpu.async_copy(table_hbm.at[idx_v], rows_v, sem).wait()

            def row_body(r, c):
                for j in range(D_MODEL // _L):
                    sl = pl.ds(j * _L, _L)
                    rows_v[r, sl] = rows_v[r, sl] * SCALE
                return c

            lax.fori_loop(0, CHUNK, row_body, 0)
            pltpu.sync_copy(rows_v, out_hbm.at[pl.ds(start, CHUNK)])
            return carry

        lax.fori_loop(0, n_chunks, chunk_body, 0)

    out = gather_scale(xf, lut)
    return out.reshape(x.shape[0], x.shape[1], D_MODEL)


# speedup vs baseline: 1.1397x; 1.1397x over previous
import functools
import math

import jax
import jax.numpy as jnp
from jax import lax
from jax.experimental import pallas as pl
from jax.experimental.pallas import tpu as pltpu
from jax.experimental.pallas import tpu_sc as plsc

D_MODEL = 64
SCALE = math.sqrt(D_MODEL)

_info = plsc.get_sparse_core_info()
_NC = _info.num_cores
_NS = _info.num_subcores
_L = _info.num_lanes
_NW = _NC * _NS

CHUNK = 400
NBUF = 4
PREFETCH = 2
ROW_UNROLL = 4


@jax.jit
def kernel(x, lut):
    B = x.shape[0] * x.shape[1]
    xf = x.reshape(B).astype(jnp.int32)
    b_per_w = B // _NW
    n_chunks = b_per_w // CHUNK
    assert b_per_w % CHUNK == 0 and n_chunks % NBUF == 0

    mesh = plsc.VectorSubcoreMesh(core_axis_name="c", subcore_axis_name="s")

    @functools.partial(
        pl.kernel,
        mesh=mesh,
        out_type=jax.ShapeDtypeStruct((B, D_MODEL), jnp.float32),
        scratch_types=[
            pltpu.VMEM((b_per_w,), jnp.int32),
            [pltpu.VMEM((CHUNK, D_MODEL), jnp.float32) for _ in range(NBUF)],
            [pltpu.SemaphoreType.DMA for _ in range(NBUF)],
            [pltpu.SemaphoreType.DMA for _ in range(NBUF)],
        ],
        compiler_params=pltpu.CompilerParams(use_tc_tiling_on_sc=False),
    )
    def gather_scale(idx_hbm, table_hbm, out_hbm, idx_v, rows, sg, ss):
        wid = lax.axis_index("s") * _NC + lax.axis_index("c")
        base = wid * b_per_w

        pltpu.sync_copy(idx_hbm.at[pl.ds(base, b_per_w)], idx_v)

        def start_gather(g, b):
            pltpu.async_copy(
                table_hbm.at[idx_v.at[pl.ds(g * CHUNK, CHUNK)]], rows[b], sg[b]
            )

        def wait_store(g_prev, b):
            pltpu.make_async_copy(
                rows[b], out_hbm.at[pl.ds(base + g_prev * CHUNK, CHUNK)], ss[b]
            ).wait()

        def scale_rows(b):
            def row_body(r, c):
                r0 = r * ROW_UNROLL
                for u in range(ROW_UNROLL):
                    for j in range(D_MODEL // _L):
                        sl = pl.ds(j * _L, _L)
                        rows[b][r0 + u, sl] = rows[b][r0 + u, sl] * SCALE
                return c

            lax.fori_loop(0, CHUNK // ROW_UNROLL, row_body, 0, unroll=False)

        def process(g, b, prefetch):
            pltpu.make_async_copy(
                table_hbm.at[idx_v.at[pl.ds(g * CHUNK, CHUNK)]], rows[b], sg[b]
            ).wait()
            scale_rows(b)
            pltpu.async_copy(
                rows[b], out_hbm.at[pl.ds(base + g * CHUNK, CHUNK)], ss[b]
            )
            if prefetch:
                h = g + PREFETCH
                bh = (b + PREFETCH) % NBUF
                is_first_use = g < NBUF - PREFETCH

                @pl.when(jnp.logical_not(is_first_use))
                def _():
                    wait_store(h - NBUF, bh)

                start_gather(h, bh)

        for g in range(PREFETCH):
            start_gather(g, g % NBUF)

        def loop_body(k, c):
            g0 = k * NBUF
            for b in range(NBUF):
                process(g0 + b, b, prefetch=True)
            return c

        n_main = n_chunks // NBUF - 1
        lax.fori_loop(0, n_main, loop_body, 0, unroll=False)

        g0 = n_main * NBUF
        for b in range(NBUF):
            process(g0 + b, b, prefetch=(b < NBUF - PREFETCH))

        for b in range(NBUF):
            wait_store(g0 + b, b)

    out = gather_scale(xf, lut)
    return out.reshape(x.shape[0], x.shape[1], D_MODEL)
